# baseline (speedup 1.0000x reference)
import dataclasses
import functools

import jax
import jax.numpy as jnp
from jax import lax
from jax.experimental import pallas as pl
from jax.experimental.pallas import tpu as pltpu
from jax.experimental.pallas import tpu_sc as plsc

DF = 16
SUB = 128
NSLOT = 4
N_TILES = 16
N_CORES = 2
ROWS_BLK = 640
N_PAD = N_TILES * ROWS_BLK
DEN_ROWS = N_PAD // DF
ACC_ROWS = N_PAD + DEN_ROWS
WB_ROWS = ACC_ROWS // N_TILES


def _front_body(x_ref, w1_ref, b1_ref, hn_ref, nrm_ref):
    h = jnp.maximum(
        jnp.dot(x_ref[...], w1_ref[...], preferred_element_type=jnp.float32)
        + b1_ref[...],
        0.0,
    )
    nrm = jnp.sqrt(jnp.sum(h * h, axis=1, keepdims=True))
    hn_ref[...] = h / jnp.maximum(nrm, 1e-12)
    nrm_ref[...] = nrm


def _combine_body(f0_ref, f1_ref, d0_ref, d1_ref, x1_ref, xn_ref, nrm_ref):
    num = f0_ref[...] + f1_ref[...]
    den = d0_ref[...] + d1_ref[...]
    x1 = num / jnp.maximum(den, 1e-30)
    nrm = jnp.sqrt(jnp.sum(x1 * x1, axis=1, keepdims=True))
    x1_ref[...] = x1
    xn_ref[...] = x1 / jnp.maximum(nrm, 1e-12)
    nrm_ref[...] = nrm


def _head_body(f0_ref, f1_ref, d0_ref, d1_ref, w2_ref, b2_ref, out_ref):
    num = f0_ref[...] + f1_ref[...]
    den = d0_ref[...] + d1_ref[...]
    h2 = num / jnp.maximum(den, 1e-30)
    logits = (
        jnp.dot(h2, w2_ref[...], preferred_element_type=jnp.float32)
        + b2_ref[...]
    )
    m = jnp.max(logits, axis=1, keepdims=True)
    z = logits - m
    lse = jnp.log(jnp.sum(jnp.exp(z), axis=1, keepdims=True))
    out_ref[...] = z - lse


def _sc_prop(hnt, nrm2d, src2d, dst2d, beta16, iid2d, k_subs):
    mesh = plsc.VectorSubcoreMesh(core_axis_name="c", subcore_axis_name="s")
    cp = pltpu.CompilerParams()
    if "needs_layout_passes" in pltpu.CompilerParams.__dataclass_fields__:
        cp = dataclasses.replace(cp, needs_layout_passes=False)
    if "use_tc_tiling_on_sc" in pltpu.CompilerParams.__dataclass_fields__:
        cp = dataclasses.replace(cp, use_tc_tiling_on_sc=False)

    @functools.partial(
        pl.kernel,
        compiler_params=cp,
        out_type=jax.ShapeDtypeStruct((N_CORES, ACC_ROWS, DF), jnp.float32),
        mesh=mesh,
        scratch_types=(
            [pltpu.VMEM((SUB, DF), jnp.float32)] * NSLOT
            + [pltpu.VMEM((SUB, DF), jnp.float32)] * NSLOT
            + [pltpu.VMEM((SUB, DF), jnp.float32)] * NSLOT
            + [
                pltpu.VMEM((k_subs, SUB), jnp.int32),
                pltpu.VMEM((k_subs, SUB), jnp.int32),
                pltpu.VMEM((DF,), jnp.float32),
                pltpu.VMEM((DEN_ROWS, DF), jnp.float32),
                pltpu.VMEM((DEN_ROWS, DF), jnp.float32),
                pltpu.VMEM((DEN_ROWS // SUB, SUB), jnp.int32),
                pltpu.VMEM_SHARED((ACC_ROWS, DF), jnp.float32),
            ]
            + [pltpu.SemaphoreType.DMA] * NSLOT
            + [pltpu.SemaphoreType.DMA] * NSLOT
        ),
    )
    def prop(hnt_hbm, nrm_hbm, src_hbm, dst_hbm, beta_hbm, iid_hbm, out_hbm,
             *scratch):
        sf = list(scratch[0:NSLOT])
        tb = list(scratch[NSLOT:2 * NSLOT])
        wb = list(scratch[2 * NSLOT:3 * NSLOT])
        (sidx, didx, bbuf, denl, nrmt, iid, acc_sh) = scratch[
            3 * NSLOT:3 * NSLOT + 7]
        gsem = list(scratch[3 * NSLOT + 7:3 * NSLOT + 7 + NSLOT])
        ssem = list(scratch[3 * NSLOT + 7 + NSLOT:])
        c = lax.axis_index("c")
        s = lax.axis_index("s")
        zero16 = jnp.zeros((DF,), jnp.float32)

        for b in range(NSLOT):
            @pl.loop(0, SUB)
            def _(r):
                wb[b][r, pl.ds(0, DF)] = zero16

        @pl.loop(0, DEN_ROWS)
        def _(r):
            denl[r, pl.ds(0, DF)] = zero16

        zbase = s * WB_ROWS
        for j in range(WB_ROWS // SUB):
            pltpu.sync_copy(wb[0], acc_sh.at[pl.ds(zbase + j * SUB, SUB)])
        rem_rows = WB_ROWS % SUB
        if rem_rows:
            pltpu.sync_copy(
                wb[0].at[pl.ds(0, rem_rows)],
                acc_sh.at[pl.ds(zbase + (WB_ROWS // SUB) * SUB, rem_rows)])

        pltpu.sync_copy(beta_hbm, bbuf)
        bv = bbuf[...]
        pltpu.sync_copy(iid_hbm, iid)
        pltpu.sync_copy(nrm_hbm, nrmt)

        tile = c * N_TILES + s
        pltpu.sync_copy(src_hbm.at[pl.ds(tile * k_subs, k_subs)], sidx)
        pltpu.sync_copy(dst_hbm.at[pl.ds(tile * k_subs, k_subs)], didx)
        plsc.subcore_barrier()

        iota = lax.iota(jnp.int32, DF)

        def gather_start(j, sl):
            pltpu.async_copy(hnt_hbm.at[sidx.at[j]], sf[sl], gsem[sl])
            pltpu.async_copy(hnt_hbm.at[didx.at[j]], tb[sl], gsem[sl])

        def gather_wait(sl):
            pltpu.make_async_copy(
                hnt_hbm.at[sidx.at[0]], sf[sl], gsem[sl]).wait()
            pltpu.make_async_copy(
                hnt_hbm.at[didx.at[0]], tb[sl], gsem[sl]).wait()

        def scatter_start(j, sl):
            pltpu.async_copy(wb[sl], acc_sh.at[didx.at[j]], ssem[sl],
                             add=True)

        def scatter_wait(sl):
            pltpu.make_async_copy(
                wb[sl], acc_sh.at[didx.at[0]], ssem[sl]).wait()

        def compute(j, sl):
            diags = [lax.bitwise_and(iota + d, 15) for d in range(DF)]

            @pl.loop(0, SUB // DF)
            def _(g):
                rows = iota + g * DF
                accs = [zero16, zero16, zero16, zero16]
                for d in range(DF):
                    a = plsc.load_gather(sf[sl], [rows, diags[d]])
                    b = plsc.load_gather(tb[sl], [rows, diags[d]])
                    accs[d % 4] = accs[d % 4] + a * b
                acc = (accs[0] + accs[1]) + (accs[2] + accs[3])
                e = jnp.exp(acc * bv)
                s16 = sidx[j, pl.ds(g * DF, DF)]
                nv = plsc.load_gather(
                    nrmt, [lax.shift_right_logical(s16, 4),
                           lax.bitwise_and(s16, 15)])
                en = e * nv
                for d in range(DF):
                    f = plsc.load_gather(sf[sl], [rows, diags[d]])
                    plsc.store_scatter(wb[sl], [rows, diags[d]], f * en)
                d16 = didx[j, pl.ds(g * DF, DF)]
                plsc.addupdate_scatter(
                    denl, [lax.shift_right_logical(d16, 4),
                           lax.bitwise_and(d16, 15)], e)

        for q in range(NSLOT):
            scatter_start(0, q)
        for q in range(NSLOT - 1):
            gather_start(q, q)

        @pl.loop(0, k_subs // NSLOT)
        def _(i):
            a = NSLOT * i
            for q in range(NSLOT):
                gather_start(
                    lax.rem(a + q + NSLOT - 1, jnp.int32(k_subs)),
                    (q + NSLOT - 1) % NSLOT)
                scatter_wait(q)
                gather_wait(q)
                scatter_start(a + q, q)

        for q in range(NSLOT - 1):
            gather_wait(q)
        for q in range(NSLOT):
            scatter_wait(q)

        for j in range(DEN_ROWS // SUB):
            pltpu.sync_copy(denl.at[pl.ds(j * SUB, SUB)],
                            acc_sh.at[iid.at[j]], add=True)

        plsc.subcore_barrier()
        pltpu.sync_copy(
            acc_sh.at[pl.ds(s * WB_ROWS, WB_ROWS)],
            out_hbm.at[c, pl.ds(s * WB_ROWS, WB_ROWS)])

    return prop(hnt, nrm2d, src2d, dst2d, beta16, iid2d)


@jax.jit
def kernel(x, edge_index, W1, b1, beta2, W2, b2):
    n, d = x.shape
    e = edge_index.shape[1]
    grid_n = N_PAD // ROWS_BLK

    loop = jnp.arange(n, dtype=jnp.int32)
    src = jnp.concatenate([edge_index[0], loop])
    dst = jnp.concatenate([edge_index[1], loop])
    e_tot = e + n
    blk_edges = N_CORES * N_TILES * SUB
    k_subs = -(-e_tot // blk_edges)
    k_subs += (-k_subs) % NSLOT
    e_pad = k_subs * blk_edges
    pad = e_pad - e_tot
    src2d = jnp.concatenate(
        [src, jnp.full((pad,), n, jnp.int32)]).reshape(-1, SUB)
    dst2d = jnp.concatenate(
        [dst, jnp.full((pad,), n, jnp.int32)]).reshape(-1, SUB)
    iid2d = (N_PAD + jnp.arange(DEN_ROWS, dtype=jnp.int32)).reshape(-1, SUB)
    x_pad = jnp.pad(x, ((0, N_PAD - n), (0, 0)))

    hn_h, nrm_h = pl.pallas_call(
        _front_body,
        grid=(grid_n,),
        in_specs=[
            pl.BlockSpec((ROWS_BLK, d), lambda i: (i, 0)),
            pl.BlockSpec((d, DF), lambda i: (0, 0)),
            pl.BlockSpec((1, DF), lambda i: (0, 0)),
        ],
        out_specs=[
            pl.BlockSpec((ROWS_BLK, DF), lambda i: (i, 0)),
            pl.BlockSpec((ROWS_BLK, 1), lambda i: (i, 0)),
        ],
        out_shape=[
            jax.ShapeDtypeStruct((N_PAD, DF), jnp.float32),
            jax.ShapeDtypeStruct((N_PAD, 1), jnp.float32),
        ],
    )(x_pad, W1, b1.reshape(1, DF))

    acc1 = _sc_prop(hn_h, nrm_h.reshape(DEN_ROWS, DF), src2d, dst2d,
                    jnp.ones((DF,), jnp.float32), iid2d, k_subs)
    f1_0 = acc1[0, :N_PAD]
    f1_1 = acc1[1, :N_PAD]
    d1_0 = acc1[0, N_PAD:].reshape(N_PAD, 1)
    d1_1 = acc1[1, N_PAD:].reshape(N_PAD, 1)

    x1_pad, x1n, nrm_x1 = pl.pallas_call(
        _combine_body,
        grid=(grid_n,),
        in_specs=[
            pl.BlockSpec((ROWS_BLK, DF), lambda i: (i, 0)),
            pl.BlockSpec((ROWS_BLK, DF), lambda i: (i, 0)),
            pl.BlockSpec((ROWS_BLK, 1), lambda i: (i, 0)),
            pl.BlockSpec((ROWS_BLK, 1), lambda i: (i, 0)),
        ],
        out_specs=[
            pl.BlockSpec((ROWS_BLK, DF), lambda i: (i, 0)),
            pl.BlockSpec((ROWS_BLK, DF), lambda i: (i, 0)),
            pl.BlockSpec((ROWS_BLK, 1), lambda i: (i, 0)),
        ],
        out_shape=[
            jax.ShapeDtypeStruct((N_PAD, DF), jnp.float32),
            jax.ShapeDtypeStruct((N_PAD, DF), jnp.float32),
            jax.ShapeDtypeStruct((N_PAD, 1), jnp.float32),
        ],
    )(f1_0, f1_1, d1_0, d1_1)

    acc2 = _sc_prop(x1n, nrm_x1.reshape(DEN_ROWS, DF), src2d, dst2d,
                    jnp.full((DF,), beta2, jnp.float32), iid2d, k_subs)
    f2_0 = acc2[0, :N_PAD]
    f2_1 = acc2[1, :N_PAD]
    d2_0 = acc2[0, N_PAD:].reshape(N_PAD, 1)
    d2_1 = acc2[1, N_PAD:].reshape(N_PAD, 1)

    logp = pl.pallas_call(
        _head_body,
        grid=(grid_n,),
        in_specs=[
            pl.BlockSpec((ROWS_BLK, DF), lambda i: (i, 0)),
            pl.BlockSpec((ROWS_BLK, DF), lambda i: (i, 0)),
            pl.BlockSpec((ROWS_BLK, 1), lambda i: (i, 0)),
            pl.BlockSpec((ROWS_BLK, 1), lambda i: (i, 0)),
            pl.BlockSpec((DF, 10), lambda i: (0, 0)),
            pl.BlockSpec((1, 10), lambda i: (0, 0)),
        ],
        out_specs=pl.BlockSpec((ROWS_BLK, 10), lambda i: (i, 0)),
        out_shape=jax.ShapeDtypeStruct((N_PAD, 10), jnp.float32),
    )(f2_0, f2_1, d2_0, d2_1, W2, b2.reshape(1, 10))

    return logp[:n], x1_pad[:n]


# speedup vs baseline: 1.2408x; 1.2408x over previous
import dataclasses
import functools

import jax
import jax.numpy as jnp
from jax import lax
from jax.experimental import pallas as pl
from jax.experimental.pallas import tpu as pltpu
from jax.experimental.pallas import tpu_sc as plsc

DF = 16
SUB = 128
NSLOT = 4
N_TILES = 16
N_CORES = 2
ROWS_BLK = 640
N_PAD = N_TILES * ROWS_BLK
DEN_ROWS = N_PAD // DF
ACC_ROWS = N_PAD + DEN_ROWS
WB_ROWS = ACC_ROWS // N_TILES


def _front_body(x_ref, w1_ref, b1_ref, hn_ref, nrm_ref):
    h = jnp.maximum(
        jnp.dot(x_ref[...], w1_ref[...], preferred_element_type=jnp.float32)
        + b1_ref[...],
        0.0,
    )
    nrm = jnp.sqrt(jnp.sum(h * h, axis=1, keepdims=True))
    hn_ref[...] = h / jnp.maximum(nrm, 1e-12)
    nrm_ref[...] = nrm


def _combine_body(f0_ref, f1_ref, d0_ref, d1_ref, x1_ref, xn_ref, nrm_ref):
    num = f0_ref[0] + f1_ref[0]
    den = d0_ref[...] + d1_ref[...]
    x1 = num / jnp.maximum(den, 1e-30)
    nrm = jnp.sqrt(jnp.sum(x1 * x1, axis=1, keepdims=True))
    x1_ref[...] = x1
    xn_ref[...] = x1 / jnp.maximum(nrm, 1e-12)
    nrm_ref[...] = nrm


def _head_body(f0_ref, f1_ref, d0_ref, d1_ref, w2_ref, b2_ref, out_ref):
    num = f0_ref[0] + f1_ref[0]
    den = d0_ref[...] + d1_ref[...]
    h2 = num / jnp.maximum(den, 1e-30)
    logits = (
        jnp.dot(h2, w2_ref[...], preferred_element_type=jnp.float32)
        + b2_ref[...]
    )
    m = jnp.max(logits, axis=1, keepdims=True)
    z = logits - m
    lse = jnp.log(jnp.sum(jnp.exp(z), axis=1, keepdims=True))
    out_ref[...] = z - lse


def _sc_prop(hnt, nrm2d, src2d, dst2d, beta16, iid2d, k_subs):
    mesh = plsc.VectorSubcoreMesh(core_axis_name="c", subcore_axis_name="s")
    cp = pltpu.CompilerParams()
    if "needs_layout_passes" in pltpu.CompilerParams.__dataclass_fields__:
        cp = dataclasses.replace(cp, needs_layout_passes=False)
    if "use_tc_tiling_on_sc" in pltpu.CompilerParams.__dataclass_fields__:
        cp = dataclasses.replace(cp, use_tc_tiling_on_sc=False)

    @functools.partial(
        pl.kernel,
        compiler_params=cp,
        out_type=jax.ShapeDtypeStruct((N_CORES, ACC_ROWS, DF), jnp.float32),
        mesh=mesh,
        scratch_types=(
            [pltpu.VMEM((SUB, DF), jnp.float32)] * NSLOT
            + [pltpu.VMEM((SUB, DF), jnp.float32)] * NSLOT
            + [pltpu.VMEM((SUB, DF), jnp.float32)] * NSLOT
            + [
                pltpu.VMEM((k_subs, SUB), jnp.int32),
                pltpu.VMEM((k_subs, SUB), jnp.int32),
                pltpu.VMEM((DF,), jnp.float32),
                pltpu.VMEM((DEN_ROWS, DF), jnp.float32),
                pltpu.VMEM((DEN_ROWS, DF), jnp.float32),
                pltpu.VMEM((DEN_ROWS // SUB, SUB), jnp.int32),
                pltpu.VMEM_SHARED((ACC_ROWS, DF), jnp.float32),
            ]
            + [pltpu.SemaphoreType.DMA] * NSLOT
            + [pltpu.SemaphoreType.DMA] * NSLOT
        ),
    )
    def prop(hnt_hbm, nrm_hbm, src_hbm, dst_hbm, beta_hbm, iid_hbm, out_hbm,
             *scratch):
        sf = list(scratch[0:NSLOT])
        tb = list(scratch[NSLOT:2 * NSLOT])
        wb = list(scratch[2 * NSLOT:3 * NSLOT])
        (sidx, didx, bbuf, denl, nrmt, iid, acc_sh) = scratch[
            3 * NSLOT:3 * NSLOT + 7]
        gsem = list(scratch[3 * NSLOT + 7:3 * NSLOT + 7 + NSLOT])
        ssem = list(scratch[3 * NSLOT + 7 + NSLOT:])
        c = lax.axis_index("c")
        s = lax.axis_index("s")
        zero16 = jnp.zeros((DF,), jnp.float32)

        for b in range(NSLOT):
            @pl.loop(0, SUB)
            def _(r):
                wb[b][r, pl.ds(0, DF)] = zero16

        @pl.loop(0, DEN_ROWS)
        def _(r):
            denl[r, pl.ds(0, DF)] = zero16

        zbase = s * WB_ROWS
        for j in range(WB_ROWS // SUB):
            pltpu.sync_copy(wb[0], acc_sh.at[pl.ds(zbase + j * SUB, SUB)])
        rem_rows = WB_ROWS % SUB
        if rem_rows:
            pltpu.sync_copy(
                wb[0].at[pl.ds(0, rem_rows)],
                acc_sh.at[pl.ds(zbase + (WB_ROWS // SUB) * SUB, rem_rows)])

        pltpu.sync_copy(beta_hbm, bbuf)
        bv = bbuf[...]
        pltpu.sync_copy(iid_hbm, iid)
        pltpu.sync_copy(nrm_hbm, nrmt)

        tile = c * N_TILES + s
        pltpu.sync_copy(src_hbm.at[pl.ds(tile * k_subs, k_subs)], sidx)
        pltpu.sync_copy(dst_hbm.at[pl.ds(tile * k_subs, k_subs)], didx)
        plsc.subcore_barrier()

        iota = lax.iota(jnp.int32, DF)

        def gather_start(j, sl):
            pltpu.async_copy(hnt_hbm.at[sidx.at[j]], sf[sl], gsem[sl])
            pltpu.async_copy(hnt_hbm.at[didx.at[j]], tb[sl], gsem[sl])

        def gather_wait(sl):
            pltpu.make_async_copy(
                hnt_hbm.at[sidx.at[0]], sf[sl], gsem[sl]).wait()
            pltpu.make_async_copy(
                hnt_hbm.at[didx.at[0]], tb[sl], gsem[sl]).wait()

        def scatter_start(j, sl):
            pltpu.async_copy(wb[sl], acc_sh.at[didx.at[j]], ssem[sl],
                             add=True)

        def scatter_wait(sl):
            pltpu.make_async_copy(
                wb[sl], acc_sh.at[didx.at[0]], ssem[sl]).wait()

        def compute(j, sl):
            diags = [lax.bitwise_and(iota + d, 15) for d in range(DF)]

            @pl.loop(0, SUB // DF)
            def _(g):
                rows = iota + g * DF
                accs = [zero16, zero16, zero16, zero16]
                for d in range(DF):
                    a = plsc.load_gather(sf[sl], [rows, diags[d]])
                    b = plsc.load_gather(tb[sl], [rows, diags[d]])
                    accs[d % 4] = accs[d % 4] + a * b
                acc = (accs[0] + accs[1]) + (accs[2] + accs[3])
                e = jnp.exp(acc * bv)
                s16 = sidx[j, pl.ds(g * DF, DF)]
                nv = plsc.load_gather(
                    nrmt, [lax.shift_right_logical(s16, 4),
                           lax.bitwise_and(s16, 15)])
                en = e * nv
                for d in range(DF):
                    f = plsc.load_gather(sf[sl], [rows, diags[d]])
                    plsc.store_scatter(wb[sl], [rows, diags[d]], f * en)
                d16 = didx[j, pl.ds(g * DF, DF)]
                plsc.addupdate_scatter(
                    denl, [lax.shift_right_logical(d16, 4),
                           lax.bitwise_and(d16, 15)], e)

        for q in range(NSLOT):
            scatter_start(0, q)
        for q in range(NSLOT - 1):
            gather_start(q, q)

        @pl.loop(0, k_subs // NSLOT)
        def _(i):
            a = NSLOT * i
            for q in range(NSLOT):
                gather_start(
                    lax.rem(a + q + NSLOT - 1, jnp.int32(k_subs)),
                    (q + NSLOT - 1) % NSLOT)
                scatter_wait(q)
                gather_wait(q)
                compute(a + q, q)
                scatter_start(a + q, q)

        for q in range(NSLOT - 1):
            gather_wait(q)
        for q in range(NSLOT):
            scatter_wait(q)

        for j in range(DEN_ROWS // SUB):
            pltpu.sync_copy(denl.at[pl.ds(j * SUB, SUB)],
                            acc_sh.at[iid.at[j]], add=True)

        plsc.subcore_barrier()
        pltpu.sync_copy(
            acc_sh.at[pl.ds(s * WB_ROWS, WB_ROWS)],
            out_hbm.at[c, pl.ds(s * WB_ROWS, WB_ROWS)])

    return prop(hnt, nrm2d, src2d, dst2d, beta16, iid2d)


@jax.jit
def kernel(x, edge_index, W1, b1, beta2, W2, b2):
    n, d = x.shape
    e = edge_index.shape[1]
    grid_n = N_PAD // ROWS_BLK

    loop = jnp.arange(n, dtype=jnp.int32)
    src = jnp.concatenate([edge_index[0], loop])
    dst = jnp.concatenate([edge_index[1], loop])
    e_tot = e + n
    blk_edges = N_CORES * N_TILES * SUB
    k_subs = -(-e_tot // blk_edges)
    k_subs += (-k_subs) % NSLOT
    e_pad = k_subs * blk_edges
    pad = e_pad - e_tot
    src2d = jnp.concatenate(
        [src, jnp.full((pad,), n, jnp.int32)]).reshape(-1, SUB)
    dst2d = jnp.concatenate(
        [dst, jnp.full((pad,), n, jnp.int32)]).reshape(-1, SUB)
    iid2d = (N_PAD + jnp.arange(DEN_ROWS, dtype=jnp.int32)).reshape(-1, SUB)

    fr_blk = 400
    hn_h, nrm_h = pl.pallas_call(
        _front_body,
        grid=(n // fr_blk,),
        in_specs=[
            pl.BlockSpec((fr_blk, d), lambda i: (i, 0)),
            pl.BlockSpec((d, DF), lambda i: (0, 0)),
            pl.BlockSpec((1, DF), lambda i: (0, 0)),
        ],
        out_specs=[
            pl.BlockSpec((fr_blk, DF), lambda i: (i, 0)),
            pl.BlockSpec((fr_blk, 1), lambda i: (i, 0)),
        ],
        out_shape=[
            jax.ShapeDtypeStruct((N_PAD, DF), jnp.float32),
            jax.ShapeDtypeStruct((N_PAD, 1), jnp.float32),
        ],
    )(x, W1, b1.reshape(1, DF))

    acc1 = _sc_prop(hn_h, nrm_h.reshape(DEN_ROWS, DF), src2d, dst2d,
                    jnp.ones((DF,), jnp.float32), iid2d, k_subs)
    d1_0 = acc1[0, N_PAD:].reshape(N_PAD, 1)
    d1_1 = acc1[1, N_PAD:].reshape(N_PAD, 1)

    x1_pad, x1n, nrm_x1 = pl.pallas_call(
        _combine_body,
        grid=(grid_n,),
        in_specs=[
            pl.BlockSpec((1, ROWS_BLK, DF), lambda i: (0, i, 0)),
            pl.BlockSpec((1, ROWS_BLK, DF), lambda i: (1, i, 0)),
            pl.BlockSpec((ROWS_BLK, 1), lambda i: (i, 0)),
            pl.BlockSpec((ROWS_BLK, 1), lambda i: (i, 0)),
        ],
        out_specs=[
            pl.BlockSpec((ROWS_BLK, DF), lambda i: (i, 0)),
            pl.BlockSpec((ROWS_BLK, DF), lambda i: (i, 0)),
            pl.BlockSpec((ROWS_BLK, 1), lambda i: (i, 0)),
        ],
        out_shape=[
            jax.ShapeDtypeStruct((N_PAD, DF), jnp.float32),
            jax.ShapeDtypeStruct((N_PAD, DF), jnp.float32),
            jax.ShapeDtypeStruct((N_PAD, 1), jnp.float32),
        ],
    )(acc1, acc1, d1_0, d1_1)

    acc2 = _sc_prop(x1n, nrm_x1.reshape(DEN_ROWS, DF), src2d, dst2d,
                    jnp.full((DF,), beta2, jnp.float32), iid2d, k_subs)
    d2_0 = acc2[0, N_PAD:].reshape(N_PAD, 1)
    d2_1 = acc2[1, N_PAD:].reshape(N_PAD, 1)

    logp = pl.pallas_call(
        _head_body,
        grid=(grid_n,),
        in_specs=[
            pl.BlockSpec((1, ROWS_BLK, DF), lambda i: (0, i, 0)),
            pl.BlockSpec((1, ROWS_BLK, DF), lambda i: (1, i, 0)),
            pl.BlockSpec((ROWS_BLK, 1), lambda i: (i, 0)),
            pl.BlockSpec((ROWS_BLK, 1), lambda i: (i, 0)),
            pl.BlockSpec((DF, 10), lambda i: (0, 0)),
            pl.BlockSpec((1, 10), lambda i: (0, 0)),
        ],
        out_specs=pl.BlockSpec((ROWS_BLK, 10), lambda i: (i, 0)),
        out_shape=jax.ShapeDtypeStruct((N_PAD, 10), jnp.float32),
    )(acc2, acc2, d2_0, d2_1, W2, b2.reshape(1, 10))

    return logp[:n], x1_pad[:n]
